# baseline (speedup 1.0000x reference)
import jax
import jax.numpy as jnp
from jax.experimental import pallas as pl

N_TOKENS = 16384
D_MODEL = 2048
N_EXPERTS = 16
K = 2
BLK = 512


def _router_block(x_ref, wt_ref, b_ref, eps_ref, out_ref, idx_ref):
    logits = jnp.dot(x_ref[:], wt_ref[:], preferred_element_type=jnp.float32)
    logits = logits + b_ref[:]
    gate = logits[:, :N_EXPERTS]
    noisy_pre = logits[:, N_EXPERTS:]
    nl = gate + eps_ref[:] * jax.nn.softplus(noisy_pre)

    iota = jax.lax.broadcasted_iota(jnp.int32, nl.shape, 1)
    m1 = jnp.max(nl, axis=-1, keepdims=True)
    i1 = jnp.min(jnp.where(nl == m1, iota, N_EXPERTS), axis=-1, keepdims=True)
    masked = jnp.where(iota == i1, -jnp.inf, nl)
    m2 = jnp.max(masked, axis=-1, keepdims=True)
    i2 = jnp.min(jnp.where(masked == m2, iota, N_EXPERTS), axis=-1, keepdims=True)

    t = jnp.exp(m2 - m1)
    p1 = 1.0 / (1.0 + t)
    p2 = t * p1
    out_ref[:] = jnp.where(iota == i1, p1, jnp.where(iota == i2, p2, 0.0))
    idx_ref[:] = jnp.concatenate([i1, i2], axis=-1)


def kernel(x, Wg, bg, Wn, bn):
    wt = jnp.concatenate([Wg, Wn], axis=0).T
    b = jnp.concatenate([bg, bn], axis=0)[None, :]
    eps = jax.random.normal(jax.random.key(42), (N_TOKENS, N_EXPERTS),
                            dtype=jnp.float32)

    grid = (N_TOKENS // BLK,)
    router_out, topexperts = pl.pallas_call(
        _router_block,
        grid=grid,
        in_specs=[
            pl.BlockSpec((BLK, D_MODEL), lambda i: (i, 0)),
            pl.BlockSpec((D_MODEL, 2 * N_EXPERTS), lambda i: (0, 0)),
            pl.BlockSpec((1, 2 * N_EXPERTS), lambda i: (0, 0)),
            pl.BlockSpec((BLK, N_EXPERTS), lambda i: (i, 0)),
        ],
        out_specs=[
            pl.BlockSpec((BLK, N_EXPERTS), lambda i: (i, 0)),
            pl.BlockSpec((BLK, K), lambda i: (i, 0)),
        ],
        out_shape=[
            jax.ShapeDtypeStruct((N_TOKENS, N_EXPERTS), jnp.float32),
            jax.ShapeDtypeStruct((N_TOKENS, K), jnp.int32),
        ],
    )(x, wt, b, eps)
    return (router_out, topexperts)


# speedup vs baseline: 1.0792x; 1.0792x over previous
import jax
import jax.numpy as jnp
from jax.experimental import pallas as pl

N_TOKENS = 16384
D_MODEL = 2048
N_EXPERTS = 16
K = 2
BLK = 2048


def _router_block(x_ref, wt_ref, b_ref, eps_ref, out_ref, idx_ref):
    logits = jnp.dot(x_ref[:], wt_ref[:], preferred_element_type=jnp.float32)
    logits = logits + b_ref[:]
    gate = logits[:, :N_EXPERTS]
    noisy_pre = logits[:, N_EXPERTS:]
    nl = gate + eps_ref[:] * jax.nn.softplus(noisy_pre)

    iota = jax.lax.broadcasted_iota(jnp.int32, nl.shape, 1)
    m1 = jnp.max(nl, axis=-1, keepdims=True)
    i1 = jnp.min(jnp.where(nl == m1, iota, N_EXPERTS), axis=-1, keepdims=True)
    masked = jnp.where(iota == i1, -jnp.inf, nl)
    m2 = jnp.max(masked, axis=-1, keepdims=True)
    i2 = jnp.min(jnp.where(masked == m2, iota, N_EXPERTS), axis=-1, keepdims=True)

    t = jnp.exp(m2 - m1)
    p1 = 1.0 / (1.0 + t)
    p2 = t * p1
    out_ref[:] = jnp.where(iota == i1, p1, jnp.where(iota == i2, p2, 0.0))
    idx_ref[:] = jnp.concatenate([i1, i2], axis=-1)


def kernel(x, Wg, bg, Wn, bn):
    wt = jnp.concatenate([Wg, Wn], axis=0).T
    b = jnp.concatenate([bg, bn], axis=0)[None, :]
    eps = jax.random.normal(jax.random.key(42), (N_TOKENS, N_EXPERTS),
                            dtype=jnp.float32)

    grid = (N_TOKENS // BLK,)
    router_out, topexperts = pl.pallas_call(
        _router_block,
        grid=grid,
        in_specs=[
            pl.BlockSpec((BLK, D_MODEL), lambda i: (i, 0)),
            pl.BlockSpec((D_MODEL, 2 * N_EXPERTS), lambda i: (0, 0)),
            pl.BlockSpec((1, 2 * N_EXPERTS), lambda i: (0, 0)),
            pl.BlockSpec((BLK, N_EXPERTS), lambda i: (i, 0)),
        ],
        out_specs=[
            pl.BlockSpec((BLK, N_EXPERTS), lambda i: (i, 0)),
            pl.BlockSpec((BLK, K), lambda i: (i, 0)),
        ],
        out_shape=[
            jax.ShapeDtypeStruct((N_TOKENS, N_EXPERTS), jnp.float32),
            jax.ShapeDtypeStruct((N_TOKENS, K), jnp.int32),
        ],
    )(x, wt, b, eps)
    return (router_out, topexperts)
